# baseline (speedup 1.0000x reference)
import functools

import jax
import jax.numpy as jnp
from jax.experimental import pallas as pl
from jax.experimental.pallas import tpu as pltpu

B, C, H, W = 32, 384, 56, 56
E = 16
K = 2


BB = 4


def _fused_kernel(x0_ref, x1_ref, x2_ref, x3_ref, w1_ref, b1_ref, w2_ref, b2_ref,
                  rw_ref, idx_ref, stats_s):
    i = pl.program_id(0)
    r = jax.lax.broadcasted_iota(jnp.int32, (W, W), 0)
    c = jax.lax.broadcasted_iota(jnp.int32, (W, W), 1)
    d = r - c
    m4 = (4.0 * (d == 0).astype(jnp.float32)
          - (jnp.abs(d) == 1).astype(jnp.float32))
    for bb in range(BB):
        x_ref = (x0_ref, x1_ref, x2_ref, x3_ref)[bb]
        lb = 0
        accm = jnp.zeros((W, C), dtype=jnp.float32)
        accf = jnp.zeros((W, C), dtype=jnp.float32)
        for h in range(H):
            cur = x_ref[lb, h]
            lap = jax.lax.dot_general(
                m4, cur,
                (((1,), (0,)), ((), ())),
                preferred_element_type=jnp.float32,
            )
            if h > 0:
                lap = lap - x_ref[lb, h - 1]
            if h + 1 < H:
                lap = lap - x_ref[lb, h + 1]
            accm = accm + cur
            accf = accf + jnp.abs(lap)
        inv = jnp.float32(1.0 / (H * W))
        s_mean = jnp.sum(accm, axis=0) * inv
        s_freq = jnp.sum(accf, axis=0) * inv
        stats_s[pl.ds(i * BB + bb, 1), 0:C] = s_mean.reshape(1, C)
        stats_s[pl.ds(i * BB + bb, 1), C:2 * C] = s_freq.reshape(1, C)

    @pl.when(i == B // BB - 1)
    def _router():
        combined = stats_s[...]
        h1 = jax.lax.dot_general(
            combined, w1_ref[...],
            (((1,), (1,)), ((), ())),
            preferred_element_type=jnp.float32,
        ) + b1_ref[...]
        h1 = jnp.maximum(h1, 0.0)
        logits = jax.lax.dot_general(
            h1, w2_ref[...],
            (((1,), (1,)), ((), ())),
            preferred_element_type=jnp.float32,
        ) + b2_ref[...]

        iota = jax.lax.broadcasted_iota(jnp.int32, (B, E), 1)
        m1 = jnp.max(logits, axis=1, keepdims=True)
        i1 = jnp.min(jnp.where(logits == m1, iota, E), axis=1, keepdims=True)
        neg = jnp.float32(-3.0e38)
        masked = jnp.where(iota == i1, neg, logits)
        m2 = jnp.max(masked, axis=1, keepdims=True)
        i2 = jnp.min(jnp.where(masked == m2, iota, E), axis=1, keepdims=True)

        e2 = jnp.exp(m2 - m1)
        denom = 1.0 + e2
        p1 = 1.0 / denom
        p2 = e2 / denom

        rw = (jnp.where(iota == i1, p1, 0.0)
              + jnp.where(iota == i2, p2, 0.0))
        rw_ref[...] = rw.T
        idx_ref[...] = jnp.concatenate([i1, i2], axis=1).T


@functools.partial(jax.jit, static_argnames=("interpret",))
def _run(x, W1, b1, W2, b2, interpret=False):
    xt = jnp.transpose(x, (0, 2, 3, 1))
    rw, idx = pl.pallas_call(
        _fused_kernel,
        grid=(B // BB,),
        in_specs=[
            pl.BlockSpec((1, H, W, C), lambda i: (4 * i, 0, 0, 0)),
            pl.BlockSpec((1, H, W, C), lambda i: (4 * i + 1, 0, 0, 0)),
            pl.BlockSpec((1, H, W, C), lambda i: (4 * i + 2, 0, 0, 0)),
            pl.BlockSpec((1, H, W, C), lambda i: (4 * i + 3, 0, 0, 0)),
            pl.BlockSpec((C, 2 * C), lambda i: (0, 0)),
            pl.BlockSpec((1, C), lambda i: (0, 0)),
            pl.BlockSpec((E, C), lambda i: (0, 0)),
            pl.BlockSpec((1, E), lambda i: (0, 0)),
        ],
        out_specs=[
            pl.BlockSpec((E, B), lambda i: (0, 0)),
            pl.BlockSpec((K, B), lambda i: (0, 0)),
        ],
        out_shape=[
            jax.ShapeDtypeStruct((E, B), jnp.float32),
            jax.ShapeDtypeStruct((K, B), jnp.int32),
        ],
        scratch_shapes=[pltpu.VMEM((B, 2 * C), jnp.float32)],
        compiler_params=pltpu.CompilerParams(
            dimension_semantics=("arbitrary",),
        ),
        interpret=interpret,
    )(xt, xt, xt, xt, W1, b1.reshape(1, C), W2, b2.reshape(1, E))
    return (jnp.transpose(rw, (1, 0)).reshape(B, E, 1, 1),
            jnp.transpose(idx, (1, 0)).reshape(B, K, 1, 1))


def kernel(x, W1, b1, W2, b2):
    return _run(x, W1, b1, W2, b2)


# speedup vs baseline: 1.0417x; 1.0417x over previous
import functools

import jax
import jax.numpy as jnp
from jax.experimental import pallas as pl
from jax.experimental.pallas import tpu as pltpu

B, C, H, W = 32, 384, 56, 56
E = 16
K = 2


BB = 4


def _fused_kernel(x0_ref, x1_ref, w1_ref, b1_ref, w2_ref, b2_ref,
                  rw_ref, idx_ref, stats_s):
    i = pl.program_id(0)
    r = jax.lax.broadcasted_iota(jnp.int32, (W, W), 0)
    c = jax.lax.broadcasted_iota(jnp.int32, (W, W), 1)
    d = r - c
    m4 = (4.0 * (d == 0).astype(jnp.float32)
          - (jnp.abs(d) == 1).astype(jnp.float32))
    for bb in range(BB):
        x_ref = x0_ref if bb < BB // 2 else x1_ref
        lb = bb % (BB // 2)
        accm = jnp.zeros((W, C), dtype=jnp.float32)
        accf = jnp.zeros((W, C), dtype=jnp.float32)
        for h in range(H):
            cur = x_ref[lb, h]
            lap = jax.lax.dot_general(
                m4, cur,
                (((1,), (0,)), ((), ())),
                preferred_element_type=jnp.float32,
            )
            if h > 0:
                lap = lap - x_ref[lb, h - 1]
            if h + 1 < H:
                lap = lap - x_ref[lb, h + 1]
            accm = accm + cur
            accf = accf + jnp.abs(lap)
        inv = jnp.float32(1.0 / (H * W))
        s_mean = jnp.sum(accm, axis=0) * inv
        s_freq = jnp.sum(accf, axis=0) * inv
        stats_s[pl.ds(i * BB + bb, 1), 0:C] = s_mean.reshape(1, C)
        stats_s[pl.ds(i * BB + bb, 1), C:2 * C] = s_freq.reshape(1, C)

    @pl.when(i == B // BB - 1)
    def _router():
        combined = stats_s[...]
        h1 = jax.lax.dot_general(
            combined, w1_ref[...],
            (((1,), (1,)), ((), ())),
            preferred_element_type=jnp.float32,
        ) + b1_ref[...][None, :]
        h1 = jnp.maximum(h1, 0.0)
        logits = jax.lax.dot_general(
            h1, w2_ref[...],
            (((1,), (1,)), ((), ())),
            preferred_element_type=jnp.float32,
        ) + b2_ref[...]

        iota = jax.lax.broadcasted_iota(jnp.int32, (B, E), 1)
        m1 = jnp.max(logits, axis=1, keepdims=True)
        i1 = jnp.min(jnp.where(logits == m1, iota, E), axis=1, keepdims=True)
        neg = jnp.float32(-3.0e38)
        masked = jnp.where(iota == i1, neg, logits)
        m2 = jnp.max(masked, axis=1, keepdims=True)
        i2 = jnp.min(jnp.where(masked == m2, iota, E), axis=1, keepdims=True)

        e2 = jnp.exp(m2 - m1)
        denom = 1.0 + e2
        p1 = 1.0 / denom
        p2 = e2 / denom

        rw = (jnp.where(iota == i1, p1, 0.0)
              + jnp.where(iota == i2, p2, 0.0))
        rw_ref[...] = rw.T
        idx_ref[...] = jnp.concatenate([i1, i2], axis=1).T


@functools.partial(jax.jit, static_argnames=("interpret",))
def _run(x, W1, b1, W2, b2, interpret=False):
    xt = jnp.transpose(x, (0, 2, 3, 1))
    rw, idx = pl.pallas_call(
        _fused_kernel,
        grid=(B // BB,),
        in_specs=[
            pl.BlockSpec((BB // 2, H, W, C), lambda i: (2 * i, 0, 0, 0)),
            pl.BlockSpec((BB // 2, H, W, C), lambda i: (2 * i + 1, 0, 0, 0)),
            pl.BlockSpec((C, 2 * C), lambda i: (0, 0)),
            pl.BlockSpec((C,), lambda i: (0,)),
            pl.BlockSpec((E, C), lambda i: (0, 0)),
            pl.BlockSpec((1, E), lambda i: (0, 0)),
        ],
        out_specs=[
            pl.BlockSpec((E, B), lambda i: (0, 0)),
            pl.BlockSpec((K, B), lambda i: (0, 0)),
        ],
        out_shape=[
            jax.ShapeDtypeStruct((E, B), jnp.float32),
            jax.ShapeDtypeStruct((K, B), jnp.int32),
        ],
        scratch_shapes=[pltpu.VMEM((B, 2 * C), jnp.float32)],
        compiler_params=pltpu.CompilerParams(
            dimension_semantics=("arbitrary",),
        ),
        interpret=interpret,
    )(xt, xt, W1, b1, W2, b2.reshape(1, E))
    return (jnp.transpose(rw, (1, 0)).reshape(B, E, 1, 1),
            jnp.transpose(idx, (1, 0)).reshape(B, K, 1, 1))


def kernel(x, W1, b1, W2, b2):
    return _run(x, W1, b1, W2, b2)
